# baseline (speedup 1.0000x reference)
import jax
import jax.numpy as jnp
from jax import lax
from jax.experimental import pallas as pl
from jax.experimental.pallas import tpu as pltpu
from jax.experimental.pallas import tpu_sc as plsc

_BATCH = 16384
_FIELDS = 26
_DIM = 32
_N = _BATCH * _FIELDS
_V = 1000000
_NC = 2
_NS = 16
_NW = _NC * _NS
_B_PER_W = _N // _NW
_CHUNK = 1664
_N_CHUNKS = _B_PER_W // _CHUNK

_BC = 2048
_BT = 2048


def _tc_pack_table(dT):
    grid = (_V + _BC - 1) // _BC

    def body(in_ref, out_ref):
        out_ref[...] = in_ref[...].T

    return pl.pallas_call(
        body,
        grid=(grid,),
        in_specs=[pl.BlockSpec((_DIM, _BC), lambda i: (0, i))],
        out_specs=pl.BlockSpec((_BC, _DIM), lambda i: (i, 0)),
        out_shape=jax.ShapeDtypeStruct((_V, _DIM), jnp.float32),
    )(dT)


def _tc_pack_out(flat):
    def body(in_ref, out_ref):
        out_ref[0] = in_ref[0].T

    return pl.pallas_call(
        body,
        grid=(_FIELDS, _BATCH // _BT),
        in_specs=[pl.BlockSpec((1, _BT, _DIM), lambda f, i: (f, i, 0))],
        out_specs=pl.BlockSpec((1, _DIM, _BT), lambda f, i: (f, 0, i)),
        out_shape=jax.ShapeDtypeStruct((_FIELDS, _DIM, _BATCH), jnp.float32),
    )(flat)


def _sc_gather(table, idx_flat):
    mesh = plsc.VectorSubcoreMesh(core_axis_name="c", subcore_axis_name="s")

    @pl.kernel(
        out_type=jax.ShapeDtypeStruct((_N, _DIM), table.dtype),
        mesh=mesh,
        scratch_types=[
            pltpu.VMEM((_B_PER_W,), jnp.int32),
            pltpu.VMEM((2, _CHUNK, _DIM), jnp.float32),
            pltpu.SemaphoreType.DMA,
            pltpu.SemaphoreType.DMA,
            pltpu.SemaphoreType.DMA,
            pltpu.SemaphoreType.DMA,
        ],
        compiler_params=pltpu.CompilerParams(use_tc_tiling_on_sc=False),
    )
    def gather_kernel(table_hbm, idx_hbm, out_hbm, idx_v, rows_v,
                      gsem0, gsem1, osem0, osem1):
        wid = lax.axis_index("s") * _NC + lax.axis_index("c")
        base = wid * _B_PER_W
        gsem = (gsem0, gsem1)
        osem = (osem0, osem1)

        pltpu.sync_copy(idx_hbm.at[pl.ds(base, _B_PER_W)], idx_v)

        gathers = {}
        writes = {}
        gathers[0] = pltpu.async_copy(
            table_hbm.at[idx_v.at[pl.ds(0, _CHUNK)]], rows_v.at[0], gsem[0])
        for j in range(_N_CHUNKS):
            b = j % 2
            bn = (j + 1) % 2
            if j + 1 < _N_CHUNKS:
                if j >= 1:
                    writes[j - 1].wait()
                gathers[j + 1] = pltpu.async_copy(
                    table_hbm.at[idx_v.at[pl.ds((j + 1) * _CHUNK, _CHUNK)]],
                    rows_v.at[bn], gsem[bn])
            gathers[j].wait()
            writes[j] = pltpu.async_copy(
                rows_v.at[b], out_hbm.at[pl.ds(base + j * _CHUNK, _CHUNK)],
                osem[b])
        writes[_N_CHUNKS - 2].wait()
        writes[_N_CHUNKS - 1].wait()

    return gather_kernel(table, idx_flat)


def kernel(x, dictionary):
    table_rm = _tc_pack_table(dictionary.T)
    idx_flat = x.astype(jnp.int32).T.reshape(_N)
    gathered = _sc_gather(table_rm, idx_flat)
    out_fdb = _tc_pack_out(gathered.reshape(_FIELDS, _BATCH, _DIM))
    return out_fdb.transpose(2, 0, 1)


# speedup vs baseline: 2.0561x; 2.0561x over previous
import jax
import jax.numpy as jnp
from jax import lax
from jax.experimental import pallas as pl
from jax.experimental.pallas import tpu as pltpu
from jax.experimental.pallas import tpu_sc as plsc

_BATCH = 16384
_FIELDS = 26
_DIM = 32
_N = _BATCH * _FIELDS
_V = 1000000
_NC = 2
_NS = 16
_NW = _NC * _NS
_B_PER_W = _N // _NW
_CHUNK = 1664
_N_CHUNKS = _B_PER_W // _CHUNK

_BC = 2048
_BT = 2048
_M = 249856
_TAIL = _V - 4 * _M
_QW = _V // 4


def _tc_pack_table(dT):
    nblk = _M // _BC

    def body(i0, i1, i2, i3, out_ref):
        out_ref[...] = jnp.concatenate(
            [i0[...].T, i1[...].T, i2[...].T, i3[...].T], axis=1)

    def make_map(s):
        return lambda i: (0, s * nblk + i)

    return pl.pallas_call(
        body,
        grid=(nblk,),
        in_specs=[pl.BlockSpec((_DIM, _BC), make_map(s)) for s in range(4)],
        out_specs=pl.BlockSpec((_BC, 128), lambda i: (i, 0)),
        out_shape=jax.ShapeDtypeStruct((_QW, 128), jnp.float32),
    )(dT, dT, dT, dT)


def _tc_pack_out(flat128):
    def body(in_ref, out_ref):
        a = in_ref[0]
        out_ref[0] = jnp.concatenate(
            [a[:, 32 * s:32 * s + 32].T for s in range(4)], axis=1)

    return pl.pallas_call(
        body,
        grid=(_FIELDS, _BATCH // _BT),
        in_specs=[pl.BlockSpec((1, _BT // 4, 128), lambda f, i: (f, i, 0))],
        out_specs=pl.BlockSpec((1, _DIM, _BT), lambda f, i: (f, 0, i)),
        out_shape=jax.ShapeDtypeStruct((_FIELDS, _DIM, _BATCH), jnp.float32),
    )(flat128)


def _sc_gather(table, idx_flat):
    mesh = plsc.VectorSubcoreMesh(core_axis_name="c", subcore_axis_name="s")

    @pl.kernel(
        out_type=jax.ShapeDtypeStruct((_N, _DIM), table.dtype),
        mesh=mesh,
        scratch_types=[
            pltpu.VMEM((_B_PER_W,), jnp.int32),
            pltpu.VMEM((2, _CHUNK, _DIM), jnp.float32),
            pltpu.SemaphoreType.DMA,
            pltpu.SemaphoreType.DMA,
            pltpu.SemaphoreType.DMA,
            pltpu.SemaphoreType.DMA,
        ],
        compiler_params=pltpu.CompilerParams(use_tc_tiling_on_sc=False),
    )
    def gather_kernel(table_hbm, idx_hbm, out_hbm, idx_v, rows_v,
                      gsem0, gsem1, osem0, osem1):
        wid = lax.axis_index("s") * _NC + lax.axis_index("c")
        base = wid * _B_PER_W
        gsem = (gsem0, gsem1)
        osem = (osem0, osem1)

        pltpu.sync_copy(idx_hbm.at[pl.ds(base, _B_PER_W)], idx_v)

        gathers = {}
        writes = {}
        gathers[0] = pltpu.async_copy(
            table_hbm.at[idx_v.at[pl.ds(0, _CHUNK)]], rows_v.at[0], gsem[0])
        for j in range(_N_CHUNKS):
            b = j % 2
            bn = (j + 1) % 2
            if j + 1 < _N_CHUNKS:
                if j >= 1:
                    writes[j - 1].wait()
                gathers[j + 1] = pltpu.async_copy(
                    table_hbm.at[idx_v.at[pl.ds((j + 1) * _CHUNK, _CHUNK)]],
                    rows_v.at[bn], gsem[bn])
            gathers[j].wait()
            writes[j] = pltpu.async_copy(
                rows_v.at[b], out_hbm.at[pl.ds(base + j * _CHUNK, _CHUNK)],
                osem[b])
        writes[_N_CHUNKS - 2].wait()
        writes[_N_CHUNKS - 1].wait()

    return gather_kernel(table, idx_flat)


def kernel(x, dictionary):
    packed = _tc_pack_table(dictionary.T)
    tailp = dictionary[4 * _M:].reshape(_TAIL // 4, 128)
    packed = jax.lax.dynamic_update_slice(packed, tailp, (_M, 0))
    table_rm = packed.reshape(4 * _QW, _DIM)
    idx = (x.astype(jnp.int32).T
           .reshape(_FIELDS, _BATCH // _BT, 4, _BT // 4)
           .transpose(0, 1, 3, 2)
           .reshape(_N))
    idx = jnp.where(idx < 4 * _M, (idx % _M) * 4 + idx // _M, idx)
    gathered = _sc_gather(table_rm, idx)
    out_fdb = _tc_pack_out(gathered.reshape(_FIELDS, _BATCH * _DIM // 128, 128))
    return out_fdb.transpose(2, 0, 1)


# speedup vs baseline: 2.2961x; 1.1167x over previous
import jax
import jax.numpy as jnp
from jax import lax
from jax.experimental import pallas as pl
from jax.experimental.pallas import tpu as pltpu
from jax.experimental.pallas import tpu_sc as plsc

_BATCH = 16384
_FIELDS = 26
_DIM = 32
_N = _BATCH * _FIELDS
_V = 1000000
_NC = 2
_NS = 16
_NW = _NC * _NS
_B_PER_W = _N // _NW
_CHUNK = 1664
_N_CHUNKS = _B_PER_W // _CHUNK

_BC = 4096
_BT = 4096
_M = 249856
_TAIL = _V - 4 * _M
_QW = _V // 4


def _tc_pack_table(dT):
    nblk = _M // _BC

    def body(i0, i1, i2, i3, out_ref):
        out_ref[...] = jnp.concatenate(
            [i0[...].T, i1[...].T, i2[...].T, i3[...].T], axis=1)

    def make_map(s):
        return lambda i: (0, s * nblk + i)

    return pl.pallas_call(
        body,
        grid=(nblk,),
        in_specs=[pl.BlockSpec((_DIM, _BC), make_map(s)) for s in range(4)],
        out_specs=pl.BlockSpec((_BC, 128), lambda i: (i, 0)),
        out_shape=jax.ShapeDtypeStruct((_QW, 128), jnp.float32),
        compiler_params=pltpu.CompilerParams(
            dimension_semantics=("parallel",)),
    )(dT, dT, dT, dT)


def _tc_pack_out(flat128):
    def body(in_ref, out_ref):
        a = in_ref[0]
        out_ref[0] = jnp.concatenate(
            [a[:, 32 * s:32 * s + 32].T for s in range(4)], axis=1)

    return pl.pallas_call(
        body,
        grid=(_FIELDS, _BATCH // _BT),
        in_specs=[pl.BlockSpec((1, _BT // 4, 128), lambda f, i: (f, i, 0))],
        out_specs=pl.BlockSpec((1, _DIM, _BT), lambda f, i: (f, 0, i)),
        out_shape=jax.ShapeDtypeStruct((_FIELDS, _DIM, _BATCH), jnp.float32),
        compiler_params=pltpu.CompilerParams(
            dimension_semantics=("parallel", "parallel")),
    )(flat128)


def _sc_gather(table, idx_flat):
    mesh = plsc.VectorSubcoreMesh(core_axis_name="c", subcore_axis_name="s")

    @pl.kernel(
        out_type=jax.ShapeDtypeStruct((_N, _DIM), table.dtype),
        mesh=mesh,
        scratch_types=[
            pltpu.VMEM((_B_PER_W,), jnp.int32),
            pltpu.VMEM((2, _CHUNK, _DIM), jnp.float32),
            pltpu.SemaphoreType.DMA,
            pltpu.SemaphoreType.DMA,
            pltpu.SemaphoreType.DMA,
            pltpu.SemaphoreType.DMA,
        ],
        compiler_params=pltpu.CompilerParams(use_tc_tiling_on_sc=False),
    )
    def gather_kernel(table_hbm, idx_hbm, out_hbm, idx_v, rows_v,
                      gsem0, gsem1, osem0, osem1):
        wid = lax.axis_index("s") * _NC + lax.axis_index("c")
        base = wid * _B_PER_W
        gsem = (gsem0, gsem1)
        osem = (osem0, osem1)

        pltpu.sync_copy(idx_hbm.at[pl.ds(base, _B_PER_W)], idx_v)

        gathers = {}
        writes = {}
        gathers[0] = pltpu.async_copy(
            table_hbm.at[idx_v.at[pl.ds(0, _CHUNK)]], rows_v.at[0], gsem[0])
        for j in range(_N_CHUNKS):
            b = j % 2
            bn = (j + 1) % 2
            if j + 1 < _N_CHUNKS:
                if j >= 1:
                    writes[j - 1].wait()
                gathers[j + 1] = pltpu.async_copy(
                    table_hbm.at[idx_v.at[pl.ds((j + 1) * _CHUNK, _CHUNK)]],
                    rows_v.at[bn], gsem[bn])
            gathers[j].wait()
            writes[j] = pltpu.async_copy(
                rows_v.at[b], out_hbm.at[pl.ds(base + j * _CHUNK, _CHUNK)],
                osem[b])
        writes[_N_CHUNKS - 2].wait()
        writes[_N_CHUNKS - 1].wait()

    return gather_kernel(table, idx_flat)


def kernel(x, dictionary):
    packed = _tc_pack_table(dictionary.T)
    tailp = dictionary[4 * _M:].reshape(_TAIL // 4, 128)
    packed = jax.lax.dynamic_update_slice(packed, tailp, (_M, 0))
    table_rm = packed.reshape(4 * _QW, _DIM)
    idx = (x.astype(jnp.int32).T
           .reshape(_FIELDS, _BATCH // _BT, 4, _BT // 4)
           .transpose(0, 1, 3, 2)
           .reshape(_N))
    idx = jnp.where(idx < 4 * _M, (idx % _M) * 4 + idx // _M, idx)
    gathered = _sc_gather(table_rm, idx)
    out_fdb = _tc_pack_out(gathered.reshape(_FIELDS, _BATCH * _DIM // 128, 128))
    return out_fdb.transpose(2, 0, 1)


# speedup vs baseline: 2.3643x; 1.0297x over previous
import jax
import jax.numpy as jnp
from jax import lax
from jax.experimental import pallas as pl
from jax.experimental.pallas import tpu as pltpu
from jax.experimental.pallas import tpu_sc as plsc

_BATCH = 16384
_FIELDS = 26
_DIM = 32
_N = _BATCH * _FIELDS
_V = 1000000
_NC = 2
_NS = 16
_NW = _NC * _NS
_B_PER_W = _N // _NW
_CHUNK = 1664
_N_CHUNKS = _B_PER_W // _CHUNK

_BC = 8192
_BT = 8192
_M = 245760
_TAIL = _V - 4 * _M
_QW = _V // 4


def _tc_pack_table(dT):
    nblk = _M // _BC

    def body(i0, i1, i2, i3, out_ref):
        out_ref[...] = jnp.concatenate(
            [i0[...].T, i1[...].T, i2[...].T, i3[...].T], axis=1)

    def make_map(s):
        return lambda i: (0, s * nblk + i)

    return pl.pallas_call(
        body,
        grid=(nblk,),
        in_specs=[pl.BlockSpec((_DIM, _BC), make_map(s)) for s in range(4)],
        out_specs=pl.BlockSpec((_BC, 128), lambda i: (i, 0)),
        out_shape=jax.ShapeDtypeStruct((_QW, 128), jnp.float32),
        compiler_params=pltpu.CompilerParams(
            dimension_semantics=("parallel",)),
    )(dT, dT, dT, dT)


def _tc_pack_out(flat128):
    def body(in_ref, out_ref):
        a = in_ref[0]
        out_ref[0] = jnp.concatenate(
            [a[:, 32 * s:32 * s + 32].T for s in range(4)], axis=1)

    return pl.pallas_call(
        body,
        grid=(_FIELDS, _BATCH // _BT),
        in_specs=[pl.BlockSpec((1, _BT // 4, 128), lambda f, i: (f, i, 0))],
        out_specs=pl.BlockSpec((1, _DIM, _BT), lambda f, i: (f, 0, i)),
        out_shape=jax.ShapeDtypeStruct((_FIELDS, _DIM, _BATCH), jnp.float32),
        compiler_params=pltpu.CompilerParams(
            dimension_semantics=("parallel", "parallel")),
    )(flat128)


def _sc_gather(table, idx_flat):
    mesh = plsc.VectorSubcoreMesh(core_axis_name="c", subcore_axis_name="s")

    @pl.kernel(
        out_type=jax.ShapeDtypeStruct((_N, _DIM), table.dtype),
        mesh=mesh,
        scratch_types=[
            pltpu.VMEM((_B_PER_W,), jnp.int32),
            pltpu.VMEM((2, _CHUNK, _DIM), jnp.float32),
            pltpu.SemaphoreType.DMA,
            pltpu.SemaphoreType.DMA,
            pltpu.SemaphoreType.DMA,
            pltpu.SemaphoreType.DMA,
        ],
        compiler_params=pltpu.CompilerParams(use_tc_tiling_on_sc=False),
    )
    def gather_kernel(table_hbm, idx_hbm, out_hbm, idx_v, rows_v,
                      gsem0, gsem1, osem0, osem1):
        wid = lax.axis_index("s") * _NC + lax.axis_index("c")
        base = wid * _B_PER_W
        gsem = (gsem0, gsem1)
        osem = (osem0, osem1)

        pltpu.sync_copy(idx_hbm.at[pl.ds(base, _B_PER_W)], idx_v)

        gathers = {}
        writes = {}
        gathers[0] = pltpu.async_copy(
            table_hbm.at[idx_v.at[pl.ds(0, _CHUNK)]], rows_v.at[0], gsem[0])
        for j in range(_N_CHUNKS):
            b = j % 2
            bn = (j + 1) % 2
            if j + 1 < _N_CHUNKS:
                if j >= 1:
                    writes[j - 1].wait()
                gathers[j + 1] = pltpu.async_copy(
                    table_hbm.at[idx_v.at[pl.ds((j + 1) * _CHUNK, _CHUNK)]],
                    rows_v.at[bn], gsem[bn])
            gathers[j].wait()
            writes[j] = pltpu.async_copy(
                rows_v.at[b], out_hbm.at[pl.ds(base + j * _CHUNK, _CHUNK)],
                osem[b])
        writes[_N_CHUNKS - 2].wait()
        writes[_N_CHUNKS - 1].wait()

    return gather_kernel(table, idx_flat)


def kernel(x, dictionary):
    packed = _tc_pack_table(dictionary.T)
    tailp = dictionary[4 * _M:].reshape(_TAIL // 4, 128)
    packed = jax.lax.dynamic_update_slice(packed, tailp, (_M, 0))
    table_rm = packed.reshape(4 * _QW, _DIM)
    idx = (x.astype(jnp.int32).T
           .reshape(_FIELDS, _BATCH // _BT, 4, _BT // 4)
           .transpose(0, 1, 3, 2)
           .reshape(_N))
    idx = jnp.where(idx < 4 * _M, (idx % _M) * 4 + idx // _M, idx)
    gathered = _sc_gather(table_rm, idx)
    out_fdb = _tc_pack_out(gathered.reshape(_FIELDS, _BATCH * _DIM // 128, 128))
    return out_fdb.transpose(2, 0, 1)
